# baseline (speedup 1.0000x reference)
import jax
import jax.numpy as jnp
from jax import lax
from jax.experimental import pallas as pl
from jax.experimental.pallas import tpu as pltpu
from jax.experimental.pallas import tpu_sc as plsc

_D = 32
_B = 16384
_NC = 2
_NS = 16
_NW = _NC * _NS
_BPW = _B // _NW
_CHUNK = 128
_NCHUNK = _BPW // _CHUNK
_VR = 25000
_OUTR = _B // 4


def _blend(idx_v, msk_v, pad_v, out_v, u0, u1):
    def grp(g, carry):
        i16 = idx_v[pl.ds(g * 16, 16)]
        m16 = msk_v[pl.ds(g * 16, 16)]
        for r in range(16):
            su = (i16[r] & 3) * 32
            keep = m16[r] != 0
            row = g * 4 + r // 4
            col = (r % 4) * 32
            e0 = pad_v[g * 16 + r, pl.ds(su, 16)]
            e1 = pad_v[g * 16 + r, pl.ds(su + 16, 16)]
            out_v[row, pl.ds(col, 16)] = jnp.where(keep, e0, u0)
            out_v[row, pl.ds(col + 16, 16)] = jnp.where(keep, e1, u1)
        return carry
    lax.fori_loop(0, _BPW // 16, grp, 0)


def _body(idx_hbm, msk_hbm, mu_hbm, lv_hbm, muu_hbm, lvu_hbm,
          mu_out, lv_out,
          idx_v, vr_v, msk_v, muu_v, lvu_v, pad_v, omu_v, olv_v, sem):
    wid = lax.axis_index("s") * _NC + lax.axis_index("c")
    base = wid * _BPW

    pltpu.sync_copy(idx_hbm.at[pl.ds(base, _BPW)], idx_v)
    def mkvr(k, carry):
        vr_v[pl.ds(k * 16, 16)] = idx_v[pl.ds(k * 16, 16)] >> 2
        return carry
    lax.fori_loop(0, _BPW // 16, mkvr, 0)

    mu_copies = [
        pltpu.async_copy(mu_hbm.at[vr_v.at[pl.ds(j * _CHUNK, _CHUNK)]],
                         pad_v.at[pl.ds(j * _CHUNK, _CHUNK)], sem)
        for j in range(_NCHUNK)
    ]
    pltpu.sync_copy(msk_hbm.at[pl.ds(base, _BPW)], msk_v)
    pltpu.sync_copy(muu_hbm, muu_v)
    pltpu.sync_copy(lvu_hbm, lvu_v)
    mu_u = [muu_v[pl.ds(16 * t, 16)] for t in range(2)]
    lv_u = [lvu_v[pl.ds(16 * t, 16)] for t in range(2)]
    for c in mu_copies:
        c.wait()
    _blend(idx_v, msk_v, pad_v, omu_v, mu_u[0], mu_u[1])

    lv_copies = [
        pltpu.async_copy(lv_hbm.at[vr_v.at[pl.ds(j * _CHUNK, _CHUNK)]],
                         pad_v.at[pl.ds(j * _CHUNK, _CHUNK)], sem)
        for j in range(_NCHUNK)
    ]
    for c in lv_copies:
        c.wait()
    _blend(idx_v, msk_v, pad_v, olv_v, lv_u[0], lv_u[1])

    obase = wid * (_BPW // 4)
    pltpu.sync_copy(omu_v, mu_out.at[pl.ds(obase, _BPW // 4)])
    pltpu.sync_copy(olv_v, lv_out.at[pl.ds(obase, _BPW // 4)])


def kernel(regime_id, regime_seen_mask, mu_embedding, logvar_embedding,
           mu_unknown, logvar_unknown):
    idx = regime_id.astype(jnp.int32)
    mu_v = mu_embedding.reshape(_VR, 128)
    lv_v = logvar_embedding.reshape(_VR, 128)
    mesh = plsc.VectorSubcoreMesh(core_axis_name="c", subcore_axis_name="s")
    f = pl.kernel(
        _body,
        out_type=(jax.ShapeDtypeStruct((_OUTR, 128), jnp.float32),
                  jax.ShapeDtypeStruct((_OUTR, 128), jnp.float32)),
        mesh=mesh,
        compiler_params=pltpu.CompilerParams(use_tc_tiling_on_sc=True),
        scratch_types=[
            pltpu.VMEM((_BPW,), jnp.int32),
            pltpu.VMEM((_BPW,), jnp.int32),
            pltpu.VMEM((_BPW,), jnp.int32),
            pltpu.VMEM((_D,), jnp.float32),
            pltpu.VMEM((_D,), jnp.float32),
            pltpu.VMEM((_BPW, 128), jnp.float32),
            pltpu.VMEM((_BPW // 4, 128), jnp.float32),
            pltpu.VMEM((_BPW // 4, 128), jnp.float32),
            pltpu.SemaphoreType.DMA,
        ],
    )
    o_mu, o_lv = f(idx, regime_seen_mask, mu_v, lv_v,
                   mu_unknown, logvar_unknown)
    return (o_mu.reshape(_B, _D), o_lv.reshape(_B, _D))
